# baseline (speedup 1.0000x reference)
import dataclasses
import functools

import jax
import jax.numpy as jnp
from jax import lax
from jax.experimental import pallas as pl
from jax.experimental.pallas import tpu as pltpu
from jax.experimental.pallas import tpu_sc as plsc

D = 1024
V = 1024
N = 32 * 1024
NC = 2
NS = 16
L = 16
NW = NC * NS

N_SC = 16 * 1024
ROWS_PER_W = N_SC // NW
C = 8
NCHUNK = ROWS_PER_W // C
NBUF = 8

T = 256
NB_SC = N_SC // T
NB_TC = (N - N_SC) // T


def kernel(x, positions, pe_table):
    B, S, Dm = x.shape
    xf = x.reshape(N, D)
    idx = positions.reshape(N).astype(jnp.int32)
    pe_prep = jax.lax.bitcast_convert_type(
        pe_table.reshape(V, D // 32, 2, 16)
        .swapaxes(2, 3)
        .reshape(V, D // 2, 2)
        .astype(jnp.bfloat16),
        jnp.int32)
    pe_bf = pe_table.astype(jnp.bfloat16)
    idx3 = idx.reshape(N // T, 1, T)

    mesh = plsc.VectorSubcoreMesh(core_axis_name="c", subcore_axis_name="s")
    cp = pltpu.CompilerParams()
    if "needs_layout_passes" in pltpu.CompilerParams.__dataclass_fields__:
        cp = dataclasses.replace(cp, needs_layout_passes=False)

    @functools.partial(
        pl.kernel,
        out_type=jax.ShapeDtypeStruct((N, D), jnp.float32),
        mesh=mesh,
        compiler_params=cp,
        scratch_types=[
            pltpu.VMEM((ROWS_PER_W,), jnp.int32),
            pltpu.VMEM((NBUF, C, D // 2), jnp.int32),
            pltpu.VMEM((NBUF, C, D), jnp.float32),
            pltpu.SemaphoreType.DMA((NBUF,)),
            pltpu.SemaphoreType.DMA((NBUF,)),
            pltpu.SemaphoreType.DMA((NBUF,)),
            pltpu.SemaphoreType.DMA,
        ],
    )
    def sc_fn(x_hbm, idx_hbm, pe_hbm, out_hbm,
              idx_v, pe_v, x_v, sg, sx, so, si):
        wid = lax.axis_index("s") * NC + lax.axis_index("c")
        base = wid * ROWS_PER_W
        pltpu.async_copy(idx_hbm.at[pl.ds(base, ROWS_PER_W)], idx_v, si).wait()

        def start_in(chunk, b):
            row0 = chunk * C
            pltpu.async_copy(pe_hbm.at[idx_v.at[pl.ds(row0, C)]],
                             pe_v.at[b], sg.at[b])
            pltpu.async_copy(x_hbm.at[pl.ds(base + row0, C)],
                             x_v.at[b], sx.at[b])

        def wait_in(b):
            pltpu.make_async_copy(pe_hbm.at[idx_v.at[pl.ds(0, C)]],
                                  pe_v.at[b], sg.at[b]).wait()
            pltpu.make_async_copy(x_hbm.at[pl.ds(0, C)],
                                  x_v.at[b], sx.at[b]).wait()

        def start_out(chunk, b):
            pltpu.async_copy(x_v.at[b],
                             out_hbm.at[pl.ds(base + chunk * C, C)], so.at[b])

        def wait_out(b):
            pltpu.make_async_copy(x_v.at[b],
                                  out_hbm.at[pl.ds(0, C)], so.at[b]).wait()

        for j in range(NBUF - 1):
            start_in(j, j)

        @pl.loop(0, NCHUNK, step=NBUF)
        def _grp(g):
            for b in range(NBUF):
                cur = g + b
                wait_in(b)

                @pl.loop(0, C)
                def _row(r):
                    for wc in range(0, D // 2, L):
                        g0 = 2 * wc
                        w = pe_v[b, r, pl.ds(wc, L)]
                        lo = plsc.bitcast(w << 16, jnp.float32)
                        hi = plsc.bitcast(
                            w & jnp.int32(-65536), jnp.float32)
                        plsc.addupdate(x_v.at[b, r, pl.ds(g0, L)], lo)
                        plsc.addupdate(x_v.at[b, r, pl.ds(g0 + L, L)], hi)

                start_out(cur, b)
                bp = (b + NBUF - 1) % NBUF

                @pl.when(cur >= 1)
                def _():
                    wait_out(bp)

                @pl.when(cur + (NBUF - 1) < NCHUNK)
                def _():
                    start_in(cur + NBUF - 1, bp)

        wait_out((NCHUNK - 1) % NBUF)

    def tc_body(pos_ref, pe_ref, x_ref, o_ref):
        pos = pos_ref[0, 0, :].reshape(T, 1)
        iota = lax.broadcasted_iota(jnp.int32, (T, V), 1)
        onehot = (pos == iota).astype(jnp.bfloat16)
        pe_rows = jax.lax.dot_general(
            onehot, pe_ref[...], (((1,), (0,)), ((), ())),
            preferred_element_type=jnp.float32)
        o_ref[...] = x_ref[...] + pe_rows

    tc_fn = pl.pallas_call(
        tc_body,
        out_shape=jax.ShapeDtypeStruct((N - N_SC, D), jnp.float32),
        grid=(NB_TC,),
        in_specs=[
            pl.BlockSpec((1, 1, T), lambda i: (NB_SC + i, 0, 0)),
            pl.BlockSpec((V, D), lambda i: (0, 0)),
            pl.BlockSpec((T, D), lambda i: (NB_SC + i, 0)),
        ],
        out_specs=pl.BlockSpec((T, D), lambda i: (i, 0)),
        compiler_params=pltpu.CompilerParams(
            dimension_semantics=("arbitrary",)),
    )

    out_sc = sc_fn(xf, idx, pe_prep)
    out_tc = tc_fn(idx3, pe_bf, xf)
    out = lax.dynamic_update_slice(out_sc, out_tc, (N_SC, 0))
    return out.reshape(B, S, Dm)
